# Initial kernel scaffold; baseline (speedup 1.0000x reference)
import jax
import jax.numpy as jnp
from jax.experimental import pallas as pl


def kernel(X, nucleo_emb, pos_emb):
    raise NotImplementedError("write your pallas kernel here")


# speedup vs baseline: 4.3433x; 4.3433x over previous
import functools

import jax
import jax.numpy as jnp
from jax import lax
from jax.experimental import pallas as pl
from jax.experimental.pallas import tpu as pltpu
from jax.experimental.pallas import tpu_sc as plsc

_BATCH = 4096
_SEQ = 200
_DIM = 32
_VOCAB = 1000
_N = _BATCH * _SEQ

_NC = 2
_NS = 16
_NW = _NC * _NS

_ROWS_PER_W = _N // _NW
_CHUNK = 800
_IW = 100
_NSUB = _CHUNK // _IW
_NCHUNKS = _ROWS_PER_W // _CHUNK

assert _CHUNK % _SEQ == 0 and _ROWS_PER_W % _CHUNK == 0


def _body(x_hbm, tab_hbm, pos_hbm, out_hbm, idx_v, rows_v, pos_v, sem):
  wid = lax.axis_index("s") * _NC + lax.axis_index("c")
  base = wid * _ROWS_PER_W

  pltpu.sync_copy(pos_hbm, pos_v)

  def chunk_body(g, carry):
    off = base + g * _CHUNK

    idx_row = pl.multiple_of(off // _IW, 8)
    pltpu.sync_copy(x_hbm.at[pl.ds(idx_row, _NSUB)], idx_v)

    for j in range(_NSUB):
      pltpu.async_copy(
          tab_hbm.at[idx_v.at[j]], rows_v.at[pl.ds(j * _IW, _IW)], sem)
    for j in range(_NSUB):
      pltpu.make_async_copy(
          tab_hbm.at[idx_v.at[j]], rows_v.at[pl.ds(j * _IW, _IW)], sem).wait()

    def add_body(i, c):
      pv0 = pos_v[i, pl.ds(0, 16)]
      pv1 = pos_v[i, pl.ds(16, 16)]
      for r in range(_CHUNK // _SEQ):
        plsc.addupdate(rows_v.at[r * _SEQ + i, pl.ds(0, 16)], pv0)
        plsc.addupdate(rows_v.at[r * _SEQ + i, pl.ds(16, 16)], pv1)
      return c

    lax.fori_loop(0, _SEQ, add_body, 0, unroll=2)

    pltpu.sync_copy(rows_v, out_hbm.at[pl.ds(off, _CHUNK)])
    return carry

  lax.fori_loop(0, _NCHUNKS, chunk_body, 0)


@jax.jit
def _embed(x2d, nucleo_emb, pos_emb):
  mesh = plsc.VectorSubcoreMesh(
      core_axis_name="c", subcore_axis_name="s", num_cores=_NC,
      num_subcores=_NS)
  return pl.kernel(
      _body,
      out_type=jax.ShapeDtypeStruct((_N, _DIM), jnp.float32),
      mesh=mesh,
      compiler_params=pltpu.CompilerParams(use_tc_tiling_on_sc=False),
      scratch_types=[
          pltpu.VMEM((_NSUB, _IW), jnp.int32),
          pltpu.VMEM((_CHUNK, _DIM), jnp.float32),
          pltpu.VMEM((_SEQ, _DIM), jnp.float32),
          pltpu.SemaphoreType.DMA,
      ],
  )(x2d, nucleo_emb, pos_emb)


def kernel(X, nucleo_emb, pos_emb):
  x2d = X.reshape(_N // _IW, _IW)
  out = _embed(x2d, nucleo_emb, pos_emb)
  return out.reshape(_BATCH, _SEQ, _DIM)


# speedup vs baseline: 4.3546x; 1.0026x over previous
import functools

import jax
import jax.numpy as jnp
from jax import lax
from jax.experimental import pallas as pl
from jax.experimental.pallas import tpu as pltpu
from jax.experimental.pallas import tpu_sc as plsc

_BATCH = 4096
_SEQ = 200
_DIM = 32
_VOCAB = 1000
_N = _BATCH * _SEQ

_NC = 2
_NS = 16
_NW = _NC * _NS

_ROWS_PER_W = _N // _NW
_CHUNK = 800
_IW = 80
_NSUB = _CHUNK // _IW
_NCHUNKS = _ROWS_PER_W // _CHUNK
_OUTW = 128
_OROWS = _CHUNK * _DIM // _OUTW
_RPT = _CHUNK // _SEQ
_QS = _SEQ // _RPT

assert _CHUNK % _SEQ == 0 and _ROWS_PER_W % _CHUNK == 0


def _body(x_hbm, tab_hbm, pos_hbm, out_hbm, idx_v, stage_v, out_v, pos_v, sem):
  wid = lax.axis_index("s") * _NC + lax.axis_index("c")
  base = wid * _ROWS_PER_W

  pltpu.sync_copy(pos_hbm, pos_v)

  def chunk_body(g, carry):
    off = pl.multiple_of(base + g * _CHUNK, _CHUNK)

    pltpu.sync_copy(x_hbm.at[pl.ds(off, _CHUNK)], idx_v)

    for j in range(_NSUB):
      pltpu.async_copy(
          tab_hbm.at[idx_v.at[pl.ds(j * _IW, _IW)]],
          stage_v.at[pl.ds(j * _IW, _IW)], sem)
    for j in range(_NSUB):
      pltpu.make_async_copy(
          tab_hbm.at[idx_v.at[pl.ds(j * _IW, _IW)]],
          stage_v.at[pl.ds(j * _IW, _IW)], sem).wait()

    def add_body(s, c):
      for t in range(_RPT):
        pv0 = pos_v[_RPT * s + t, pl.ds(0, 16)]
        pv1 = pos_v[_RPT * s + t, pl.ds(16, 16)]
        for rr in range(_RPT):
          i = _RPT * s + _SEQ * rr + t
          g0 = stage_v[i, pl.ds(0, 16)]
          g1 = stage_v[i, pl.ds(16, 16)]
          out_v[s + _QS * rr, pl.ds(_DIM * t, 16)] = g0 + pv0
          out_v[s + _QS * rr, pl.ds(_DIM * t + 16, 16)] = g1 + pv1
      return c

    lax.fori_loop(0, _QS, add_body, 0, unroll=2)

    orow = pl.multiple_of(off * _DIM // _OUTW, _OROWS)
    pltpu.sync_copy(out_v, out_hbm.at[pl.ds(orow, _OROWS)])
    return carry

  lax.fori_loop(0, _NCHUNKS, chunk_body, 0)


@jax.jit
def _embed(x1d, nucleo_emb, pos_emb):
  mesh = plsc.VectorSubcoreMesh(
      core_axis_name="c", subcore_axis_name="s", num_cores=_NC,
      num_subcores=_NS)
  return pl.kernel(
      _body,
      out_type=jax.ShapeDtypeStruct((_N * _DIM // _OUTW, _OUTW), jnp.float32),
      mesh=mesh,
      compiler_params=pltpu.CompilerParams(use_tc_tiling_on_sc=False),
      scratch_types=[
          pltpu.VMEM((_CHUNK,), jnp.int32),
          pltpu.VMEM((_CHUNK, _DIM), jnp.float32),
          pltpu.VMEM((_OROWS, _OUTW), jnp.float32),
          pltpu.VMEM((_SEQ, _DIM), jnp.float32),
          pltpu.SemaphoreType.DMA,
      ],
  )(x1d, nucleo_emb, pos_emb)


def kernel(X, nucleo_emb, pos_emb):
  x1d = X.reshape(_N)
  out = _embed(x1d, nucleo_emb, pos_emb)
  return out.reshape(_BATCH, _SEQ, _DIM)
